# Initial kernel scaffold; baseline (speedup 1.0000x reference)
import functools

import jax
import jax.numpy as jnp
from jax import lax
from jax.experimental import pallas as pl
from jax.experimental.pallas import tpu as pltpu
from jax.experimental.pallas import tpu_sc as plsc

NC = 2
NS = 16
NW = NC * NS

CH = 80
LANES = 16
NORM = 100.0


def _silu(x):
    return x * jax.nn.sigmoid(x)


def _k1_body(h_ref, w1s_ref, w1t_ref, o1_ref, o2_ref):
    hb = h_ref[...]
    o1_ref[...] = jnp.dot(hb, w1s_ref[...], preferred_element_type=jnp.float32)
    o2_ref[...] = jnp.dot(hb, w1t_ref[...], preferred_element_type=jnp.float32)


def _k1(h, w1s, w1t, bn):
    n, nf = h.shape
    hid = w1s.shape[1]
    return pl.pallas_call(
        _k1_body,
        grid=(n // bn,),
        in_specs=[
            pl.BlockSpec((bn, nf), lambda i: (i, 0)),
            pl.BlockSpec((nf, hid), lambda i: (0, 0)),
            pl.BlockSpec((nf, hid), lambda i: (0, 0)),
        ],
        out_specs=[
            pl.BlockSpec((bn, hid), lambda i: (i, 0)),
            pl.BlockSpec((bn, hid), lambda i: (i, 0)),
        ],
        out_shape=[
            jax.ShapeDtypeStruct((n, hid), jnp.float32),
            jax.ShapeDtypeStruct((n, hid), jnp.float32),
        ],
    )(h, w1s, w1t)


def _sc_gather_sum(g1, g2, row2d, col2d, e, hid):
    epw = e // NW
    nch = epw // CH
    mesh = plsc.VectorSubcoreMesh(
        core_axis_name="c", subcore_axis_name="s",
        num_cores=NC, num_subcores=NS)

    def body(g1_hbm, g2_hbm, row_hbm, col_hbm, out_hbm,
             idx_r, idx_c, buf_a, buf_b, sem_a, sem_b):
        cid = lax.axis_index("c")
        sid = lax.axis_index("s")
        wid = sid * NC + cid
        pltpu.sync_copy(row_hbm.at[pl.ds(wid * nch, nch)], idx_r)
        pltpu.sync_copy(col_hbm.at[pl.ds(wid * nch, nch)], idx_c)

        def chunk(i, carry):
            off = wid * epw + i * CH
            cp_a = pltpu.async_copy(g1_hbm.at[idx_r.at[i]], buf_a, sem_a)
            cp_b = pltpu.async_copy(g2_hbm.at[idx_c.at[i]], buf_b, sem_b)
            cp_a.wait()
            cp_b.wait()

            def add_row(r, c2):
                for t in range(hid // LANES):
                    sl = pl.ds(t * LANES, LANES)
                    buf_a[r, sl] = buf_a[r, sl] + buf_b[r, sl]
                return c2

            lax.fori_loop(0, CH, add_row, 0)
            pltpu.sync_copy(buf_a, out_hbm.at[pl.ds(off, CH)])
            return carry

        lax.fori_loop(0, nch, chunk, 0)

    f = pl.kernel(
        body,
        out_type=jax.ShapeDtypeStruct((e, hid), jnp.float32),
        mesh=mesh,
        scratch_types=[
            pltpu.VMEM((nch, CH), jnp.int32),
            pltpu.VMEM((nch, CH), jnp.int32),
            pltpu.VMEM((CH, hid), jnp.float32),
            pltpu.VMEM((CH, hid), jnp.float32),
            pltpu.SemaphoreType.DMA,
            pltpu.SemaphoreType.DMA,
        ],
    )
    return f(g1, g2, row2d, col2d)


def _k3_body(s_ref, ea_ref, w1e_ref, b1_ref, w2_ref, b2_ref, o_ref):
    x = (s_ref[...]
         + jnp.dot(ea_ref[...], w1e_ref[...], preferred_element_type=jnp.float32)
         + b1_ref[...])
    x = _silu(x)
    y = jnp.dot(x, w2_ref[...], preferred_element_type=jnp.float32) + b2_ref[...]
    o_ref[...] = _silu(y)


def _k3(s, edge_attr, w1e, b1, w2, b2, be):
    e, hid = s.shape
    ea = edge_attr.shape[1]
    return pl.pallas_call(
        _k3_body,
        grid=(e // be,),
        in_specs=[
            pl.BlockSpec((be, hid), lambda i: (i, 0)),
            pl.BlockSpec((be, ea), lambda i: (i, 0)),
            pl.BlockSpec((ea, hid), lambda i: (0, 0)),
            pl.BlockSpec((1, hid), lambda i: (0, 0)),
            pl.BlockSpec((hid, hid), lambda i: (0, 0)),
            pl.BlockSpec((1, hid), lambda i: (0, 0)),
        ],
        out_specs=pl.BlockSpec((be, hid), lambda i: (i, 0)),
        out_shape=jax.ShapeDtypeStruct((e, hid), jnp.float32),
    )(s, edge_attr, w1e, b1, w2, b2)


def _sc_segsum(mij, row2d, n, e, hid):
    epw = e // NW
    nch = epw // CH
    rows_per_tile = n // NS
    zch = 125
    nz = rows_per_tile // zch
    mesh = plsc.VectorSubcoreMesh(
        core_axis_name="c", subcore_axis_name="s",
        num_cores=NC, num_subcores=NS)

    def body(mij_hbm, row_hbm, out_hbm, idx, buf, zbuf, agg):
        cid = lax.axis_index("c")
        sid = lax.axis_index("s")
        wid = sid * NC + cid

        def zrow(r, c2):
            for t in range(hid // LANES):
                zbuf[r, pl.ds(t * LANES, LANES)] = jnp.zeros((LANES,), jnp.float32)
            return c2

        lax.fori_loop(0, zch, zrow, 0)
        for t in range(nz):
            pltpu.sync_copy(zbuf, agg.at[pl.ds(sid * rows_per_tile + t * zch, zch)])
        plsc.subcore_barrier()

        pltpu.sync_copy(row_hbm.at[pl.ds(wid * nch, nch)], idx)

        def chunk(i, c2):
            off = wid * epw + i * CH
            pltpu.sync_copy(mij_hbm.at[pl.ds(off, CH)], buf)
            pltpu.sync_copy(buf, agg.at[idx.at[i]], add=True)
            return c2

        lax.fori_loop(0, nch, chunk, 0)
        plsc.subcore_barrier()

        for t in range(nz):
            r0 = sid * rows_per_tile + t * zch
            pltpu.sync_copy(agg.at[pl.ds(r0, zch)], zbuf)
            pltpu.sync_copy(zbuf, out_hbm.at[cid, pl.ds(r0, zch)])

    f = pl.kernel(
        body,
        out_type=jax.ShapeDtypeStruct((NC, n, hid), jnp.float32),
        mesh=mesh,
        scratch_types=[
            pltpu.VMEM((nch, CH), jnp.int32),
            pltpu.VMEM((CH, hid), jnp.float32),
            pltpu.VMEM((zch, hid), jnp.float32),
            pltpu.VMEM_SHARED((n, hid), jnp.float32),
        ],
    )
    return f(mij, row2d)


def _k5_body(h_ref, p_ref, w3h_ref, w3a_ref, b3_ref, w4_ref, b4_ref, o_ref):
    hb = h_ref[...]
    a = (p_ref[0] + p_ref[1]) * (1.0 / NORM)
    y = (jnp.dot(hb, w3h_ref[...], preferred_element_type=jnp.float32)
         + jnp.dot(a, w3a_ref[...], preferred_element_type=jnp.float32)
         + b3_ref[...])
    y = _silu(y)
    o_ref[...] = hb + jnp.dot(y, w4_ref[...], preferred_element_type=jnp.float32) + b4_ref[...]


def _k5(h, aggp, w3h, w3a, b3, w4, b4, bn):
    n, nf = h.shape
    hid = w3h.shape[1]
    return pl.pallas_call(
        _k5_body,
        grid=(n // bn,),
        in_specs=[
            pl.BlockSpec((bn, nf), lambda i: (i, 0)),
            pl.BlockSpec((NC, bn, hid), lambda i: (0, i, 0)),
            pl.BlockSpec((nf, hid), lambda i: (0, 0)),
            pl.BlockSpec((hid, hid), lambda i: (0, 0)),
            pl.BlockSpec((1, hid), lambda i: (0, 0)),
            pl.BlockSpec((hid, nf), lambda i: (0, 0)),
            pl.BlockSpec((1, nf), lambda i: (0, 0)),
        ],
        out_specs=pl.BlockSpec((bn, nf), lambda i: (i, 0)),
        out_shape=jax.ShapeDtypeStruct((n, nf), jnp.float32),
    )(h, aggp, w3h, w3a, b3, w4, b4)


def kernel(h, edge_index, edge_attr, W1, b1, W2, b2, W3, b3, W4, b4):
    n, nf = h.shape
    e, ea = edge_attr.shape
    hid = W2.shape[0]

    row = edge_index[0].astype(jnp.int32)
    col = edge_index[1].astype(jnp.int32)
    row2d = row.reshape(e // CH, CH)
    col2d = col.reshape(e // CH, CH)

    w1s = W1[:nf]
    w1t = W1[nf:2 * nf]
    w1e = W1[2 * nf:]

    g1, g2 = _k1(h, w1s, w1t, 1000)
    s = _sc_gather_sum(g1, g2, row2d, col2d, e, hid)
    mij = _k3(s, edge_attr, w1e, b1.reshape(1, hid), W2, b2.reshape(1, hid), 2000)
    aggp = _sc_segsum(mij, row2d, n, e, hid)
    h_out = _k5(h, aggp, W3[:nf], W3[nf:], b3.reshape(1, hid), W4,
                b4.reshape(1, nf), 1000)
    return (h_out, mij)


# speedup vs baseline: 3.5550x; 3.5550x over previous
import functools

import jax
import jax.numpy as jnp
from jax import lax
from jax.experimental import pallas as pl
from jax.experimental.pallas import tpu as pltpu
from jax.experimental.pallas import tpu_sc as plsc

NC = 2
NS = 16
NW = NC * NS

CH = 80
LANES = 16
NORM = 100.0


def _silu(x):
    return x * jax.nn.sigmoid(x)


def _k1_body(h_ref, w1s_ref, w1t_ref, o1_ref, o2_ref):
    hb = h_ref[...]
    o1_ref[...] = jnp.dot(hb, w1s_ref[...], preferred_element_type=jnp.float32)
    o2_ref[...] = jnp.dot(hb, w1t_ref[...], preferred_element_type=jnp.float32)


def _k1(h, w1s, w1t, bn):
    n, nf = h.shape
    hid = w1s.shape[1]
    return pl.pallas_call(
        _k1_body,
        grid=(n // bn,),
        in_specs=[
            pl.BlockSpec((bn, nf), lambda i: (i, 0)),
            pl.BlockSpec((nf, hid), lambda i: (0, 0)),
            pl.BlockSpec((nf, hid), lambda i: (0, 0)),
        ],
        out_specs=[
            pl.BlockSpec((bn, hid), lambda i: (i, 0)),
            pl.BlockSpec((bn, hid), lambda i: (i, 0)),
        ],
        out_shape=[
            jax.ShapeDtypeStruct((n, hid), jnp.float32),
            jax.ShapeDtypeStruct((n, hid), jnp.float32),
        ],
    )(h, w1s, w1t)


def _sc_gather_sum(g1, g2, row2d, col2d, e, hid):
    epw = e // NW
    nch = epw // CH
    mesh = plsc.VectorSubcoreMesh(
        core_axis_name="c", subcore_axis_name="s",
        num_cores=NC, num_subcores=NS)

    def body(g1_hbm, g2_hbm, row_hbm, col_hbm, out_hbm,
             idx_r, idx_c, buf_a, buf_b, sem_a, sem_b):
        cid = lax.axis_index("c")
        sid = lax.axis_index("s")
        wid = sid * NC + cid
        pltpu.sync_copy(row_hbm.at[wid], idx_r)
        pltpu.sync_copy(col_hbm.at[wid], idx_c)

        def chunk(i, carry):
            off = wid * epw + i * CH
            cp_a = pltpu.async_copy(g1_hbm.at[idx_r.at[i]], buf_a, sem_a)
            cp_b = pltpu.async_copy(g2_hbm.at[idx_c.at[i]], buf_b, sem_b)
            cp_a.wait()
            cp_b.wait()

            def add_row(r, c2):
                for t in range(hid // LANES):
                    sl = pl.ds(t * LANES, LANES)
                    buf_a[r, sl] = buf_a[r, sl] + buf_b[r, sl]
                return c2

            lax.fori_loop(0, CH, add_row, 0)
            pltpu.sync_copy(buf_a, out_hbm.at[pl.ds(off, CH)])
            return carry

        lax.fori_loop(0, nch, chunk, 0)

    f = pl.kernel(
        body,
        out_type=jax.ShapeDtypeStruct((e, hid), jnp.float32),
        mesh=mesh,
        scratch_types=[
            pltpu.VMEM((nch, CH), jnp.int32),
            pltpu.VMEM((nch, CH), jnp.int32),
            pltpu.VMEM((CH, hid), jnp.float32),
            pltpu.VMEM((CH, hid), jnp.float32),
            pltpu.SemaphoreType.DMA,
            pltpu.SemaphoreType.DMA,
        ],
    )
    return f(g1, g2, row2d, col2d)


def _k3_body(s_ref, ea_ref, w1e_ref, b1_ref, w2_ref, b2_ref, o_ref):
    x = (s_ref[...]
         + jnp.dot(ea_ref[...], w1e_ref[...], preferred_element_type=jnp.float32)
         + b1_ref[...])
    x = _silu(x)
    y = jnp.dot(x, w2_ref[...], preferred_element_type=jnp.float32) + b2_ref[...]
    o_ref[...] = _silu(y)


def _k3(s, edge_attr, w1e, b1, w2, b2, be):
    e, hid = s.shape
    ea = edge_attr.shape[1]
    return pl.pallas_call(
        _k3_body,
        grid=(e // be,),
        in_specs=[
            pl.BlockSpec((be, hid), lambda i: (i, 0)),
            pl.BlockSpec((be, ea), lambda i: (i, 0)),
            pl.BlockSpec((ea, hid), lambda i: (0, 0)),
            pl.BlockSpec((1, hid), lambda i: (0, 0)),
            pl.BlockSpec((hid, hid), lambda i: (0, 0)),
            pl.BlockSpec((1, hid), lambda i: (0, 0)),
        ],
        out_specs=pl.BlockSpec((be, hid), lambda i: (i, 0)),
        out_shape=jax.ShapeDtypeStruct((e, hid), jnp.float32),
    )(s, edge_attr, w1e, b1, w2, b2)


def _sc_segsum(mij, row3d, zeros_nh, n, e, hid):
    epw = e // NW
    nch = epw // CH
    mesh = plsc.VectorSubcoreMesh(
        core_axis_name="c", subcore_axis_name="s",
        num_cores=NC, num_subcores=NS)

    def body(mij_hbm, row_hbm, z_hbm, out_hbm, idx, buf, agg):
        cid = lax.axis_index("c")
        sid = lax.axis_index("s")
        wid = sid * NC + cid

        @pl.when(sid == 0)
        def _zero():
            pltpu.sync_copy(z_hbm, agg)

        plsc.subcore_barrier()

        pltpu.sync_copy(row_hbm.at[wid], idx)

        def chunk(i, c2):
            off = wid * epw + i * CH
            pltpu.sync_copy(mij_hbm.at[pl.ds(off, CH)], buf)
            pltpu.sync_copy(buf, agg.at[idx.at[i]], add=True)
            return c2

        lax.fori_loop(0, nch, chunk, 0)
        plsc.subcore_barrier()

        @pl.when(sid == 0)
        def _out():
            pltpu.sync_copy(agg, out_hbm.at[cid])

    f = pl.kernel(
        body,
        out_type=jax.ShapeDtypeStruct((NC, n, hid), jnp.float32),
        mesh=mesh,
        scratch_types=[
            pltpu.VMEM((nch, CH), jnp.int32),
            pltpu.VMEM((CH, hid), jnp.float32),
            pltpu.VMEM_SHARED((n, hid), jnp.float32),
        ],
    )
    return f(mij, row3d, zeros_nh)


def _k5_body(h_ref, p_ref, w3h_ref, w3a_ref, b3_ref, w4_ref, b4_ref, o_ref):
    hb = h_ref[...]
    a = (p_ref[0] + p_ref[1]) * (1.0 / NORM)
    y = (jnp.dot(hb, w3h_ref[...], preferred_element_type=jnp.float32)
         + jnp.dot(a, w3a_ref[...], preferred_element_type=jnp.float32)
         + b3_ref[...])
    y = _silu(y)
    o_ref[...] = hb + jnp.dot(y, w4_ref[...], preferred_element_type=jnp.float32) + b4_ref[...]


def _k5(h, aggp, w3h, w3a, b3, w4, b4, bn):
    n, nf = h.shape
    hid = w3h.shape[1]
    return pl.pallas_call(
        _k5_body,
        grid=(n // bn,),
        in_specs=[
            pl.BlockSpec((bn, nf), lambda i: (i, 0)),
            pl.BlockSpec((NC, bn, hid), lambda i: (0, i, 0)),
            pl.BlockSpec((nf, hid), lambda i: (0, 0)),
            pl.BlockSpec((hid, hid), lambda i: (0, 0)),
            pl.BlockSpec((1, hid), lambda i: (0, 0)),
            pl.BlockSpec((hid, nf), lambda i: (0, 0)),
            pl.BlockSpec((1, nf), lambda i: (0, 0)),
        ],
        out_specs=pl.BlockSpec((bn, nf), lambda i: (i, 0)),
        out_shape=jax.ShapeDtypeStruct((n, nf), jnp.float32),
    )(h, aggp, w3h, w3a, b3, w4, b4)


def kernel(h, edge_index, edge_attr, W1, b1, W2, b2, W3, b3, W4, b4):
    n, nf = h.shape
    e, ea = edge_attr.shape
    hid = W2.shape[0]

    row = edge_index[0].astype(jnp.int32)
    col = edge_index[1].astype(jnp.int32)
    nch = e // (NW * CH)
    row3d = row.reshape(NW, nch, CH)
    col3d = col.reshape(NW, nch, CH)

    w1s = W1[:nf]
    w1t = W1[nf:2 * nf]
    w1e = W1[2 * nf:]

    g1, g2 = _k1(h, w1s, w1t, 1000)
    s = _sc_gather_sum(g1, g2, row3d, col3d, e, hid)
    mij = _k3(s, edge_attr, w1e, b1.reshape(1, hid), W2, b2.reshape(1, hid), 2000)
    aggp = _sc_segsum(mij, row3d, jnp.zeros((n, hid), jnp.float32), n, e, hid)
    h_out = _k5(h, aggp, W3[:nf], W3[nf:], b3.reshape(1, hid), W4,
                b4.reshape(1, nf), 1000)
    return (h_out, mij)


# speedup vs baseline: 4.6530x; 1.3089x over previous
import functools

import jax
import jax.numpy as jnp
from jax import lax
from jax.experimental import pallas as pl
from jax.experimental.pallas import tpu as pltpu
from jax.experimental.pallas import tpu_sc as plsc

NC = 2
NS = 16
NW = NC * NS

CH = 80
LANES = 16
NORM = 100.0


def _silu(x):
    return x * jax.nn.sigmoid(x)


def _k1_body(h_ref, w1s_ref, w1t_ref, o1_ref, o2_ref):
    hb = h_ref[...]
    o1_ref[...] = jnp.dot(hb, w1s_ref[...], preferred_element_type=jnp.float32)
    o2_ref[...] = jnp.dot(hb, w1t_ref[...], preferred_element_type=jnp.float32)


def _k1(h, w1s, w1t, bn):
    n, nf = h.shape
    hid = w1s.shape[1]
    return pl.pallas_call(
        _k1_body,
        grid=(n // bn,),
        in_specs=[
            pl.BlockSpec((bn, nf), lambda i: (i, 0)),
            pl.BlockSpec((nf, hid), lambda i: (0, 0)),
            pl.BlockSpec((nf, hid), lambda i: (0, 0)),
        ],
        out_specs=[
            pl.BlockSpec((bn, hid), lambda i: (i, 0)),
            pl.BlockSpec((bn, hid), lambda i: (i, 0)),
        ],
        out_shape=[
            jax.ShapeDtypeStruct((n, hid), jnp.float32),
            jax.ShapeDtypeStruct((n, hid), jnp.float32),
        ],
    )(h, w1s, w1t)


def _sc_gather_sum(g1, g2, row3d, col3d, e, hid):
    epw = e // NW
    nch = epw // CH
    mesh = plsc.VectorSubcoreMesh(
        core_axis_name="c", subcore_axis_name="s",
        num_cores=NC, num_subcores=NS)

    assert nch % 2 == 1 and nch >= 3
    npair = (nch - 1) // 2

    def body(g1_hbm, g2_hbm, row_hbm, col_hbm, out_hbm,
             idx_r, idx_c, ba0, bb0, ba1, bb1, sg0, sg1, so0, so1):
        cid = lax.axis_index("c")
        sid = lax.axis_index("s")
        wid = sid * NC + cid
        base = wid * epw
        pltpu.sync_copy(row_hbm.at[wid], idx_r)
        pltpu.sync_copy(col_hbm.at[wid], idx_c)

        def fire(c, ba, bb, sg):
            pltpu.async_copy(g1_hbm.at[idx_r.at[c]], ba, sg)
            pltpu.async_copy(g2_hbm.at[idx_c.at[c]], bb, sg)

        def wait_gather(c, ba, bb, sg):
            pltpu.make_async_copy(g1_hbm.at[idx_r.at[c]], ba, sg).wait()
            pltpu.make_async_copy(g2_hbm.at[idx_c.at[c]], bb, sg).wait()

        def add(ba, bb):
            def add_row(r, c2):
                for t in range(hid // LANES):
                    sl = pl.ds(t * LANES, LANES)
                    ba[r, sl] = ba[r, sl] + bb[r, sl]
                return c2

            lax.fori_loop(0, CH, add_row, 0)

        def store(c, ba, so):
            pltpu.async_copy(ba, out_hbm.at[pl.ds(base + c * CH, CH)], so)

        def wait_store(c, ba, so):
            pltpu.make_async_copy(ba, out_hbm.at[pl.ds(base + c * CH, CH)], so).wait()

        fire(0, ba0, bb0, sg0)

        def pair(i2, carry):
            c0 = 2 * i2
            c1 = c0 + 1
            fire(c1, ba1, bb1, sg1)
            wait_gather(c0, ba0, bb0, sg0)

            @pl.when(i2 > 0)
            def _w0():
                wait_store(c0 - 2, ba0, so0)

            add(ba0, bb0)
            store(c0, ba0, so0)

            fire(c0 + 2, ba0, bb0, sg0)
            wait_gather(c1, ba1, bb1, sg1)

            @pl.when(i2 > 0)
            def _w1():
                wait_store(c1 - 2, ba1, so1)

            add(ba1, bb1)
            store(c1, ba1, so1)
            return carry

        lax.fori_loop(0, npair, pair, 0)

        c_last = nch - 1
        wait_gather(c_last, ba0, bb0, sg0)
        wait_store(c_last - 2, ba0, so0)
        add(ba0, bb0)
        store(c_last, ba0, so0)
        wait_store(c_last - 1, ba1, so1)
        wait_store(c_last, ba0, so0)

    f = pl.kernel(
        body,
        out_type=jax.ShapeDtypeStruct((e, hid), jnp.float32),
        mesh=mesh,
        scratch_types=[
            pltpu.VMEM((nch, CH), jnp.int32),
            pltpu.VMEM((nch, CH), jnp.int32),
            pltpu.VMEM((CH, hid), jnp.float32),
            pltpu.VMEM((CH, hid), jnp.float32),
            pltpu.VMEM((CH, hid), jnp.float32),
            pltpu.VMEM((CH, hid), jnp.float32),
            pltpu.SemaphoreType.DMA,
            pltpu.SemaphoreType.DMA,
            pltpu.SemaphoreType.DMA,
            pltpu.SemaphoreType.DMA,
        ],
    )
    return f(g1, g2, row3d, col3d)


def _k3_body(s_ref, ea_ref, w1e_ref, b1_ref, w2_ref, b2_ref, o_ref):
    x = (s_ref[...]
         + jnp.dot(ea_ref[...], w1e_ref[...], preferred_element_type=jnp.float32)
         + b1_ref[...])
    x = _silu(x)
    y = jnp.dot(x, w2_ref[...], preferred_element_type=jnp.float32) + b2_ref[...]
    o_ref[...] = _silu(y)


def _k3(s, edge_attr, w1e, b1, w2, b2, be):
    e, hid = s.shape
    ea = edge_attr.shape[1]
    return pl.pallas_call(
        _k3_body,
        grid=(e // be,),
        in_specs=[
            pl.BlockSpec((be, hid), lambda i: (i, 0)),
            pl.BlockSpec((be, ea), lambda i: (i, 0)),
            pl.BlockSpec((ea, hid), lambda i: (0, 0)),
            pl.BlockSpec((1, hid), lambda i: (0, 0)),
            pl.BlockSpec((hid, hid), lambda i: (0, 0)),
            pl.BlockSpec((1, hid), lambda i: (0, 0)),
        ],
        out_specs=pl.BlockSpec((be, hid), lambda i: (i, 0)),
        out_shape=jax.ShapeDtypeStruct((e, hid), jnp.float32),
    )(s, edge_attr, w1e, b1, w2, b2)


def _sc_segsum(mij, row3d, zeros_nh, n, e, hid):
    epw = e // NW
    nch = epw // CH
    assert nch % 2 == 1 and nch >= 3
    npair = (nch - 1) // 2
    mesh = plsc.VectorSubcoreMesh(
        core_axis_name="c", subcore_axis_name="s",
        num_cores=NC, num_subcores=NS)

    def body(mij_hbm, row_hbm, z_hbm, out_hbm, idx, buf0, buf1, si0, si1, agg):
        cid = lax.axis_index("c")
        sid = lax.axis_index("s")
        wid = sid * NC + cid

        @pl.when(sid == 0)
        def _zero():
            pltpu.sync_copy(z_hbm, agg)

        plsc.subcore_barrier()

        pltpu.sync_copy(row_hbm.at[wid], idx)
        base = wid * epw

        def fire_in(c, buf, si):
            pltpu.async_copy(mij_hbm.at[pl.ds(base + c * CH, CH)], buf, si)

        def wait_in(c, buf, si):
            pltpu.make_async_copy(mij_hbm.at[pl.ds(base + c * CH, CH)], buf, si).wait()

        fire_in(0, buf0, si0)

        def pair(i2, c2):
            c0 = 2 * i2
            c1 = c0 + 1
            fire_in(c1, buf1, si1)
            wait_in(c0, buf0, si0)
            pltpu.sync_copy(buf0, agg.at[idx.at[c0]], add=True)
            fire_in(c0 + 2, buf0, si0)
            wait_in(c1, buf1, si1)
            pltpu.sync_copy(buf1, agg.at[idx.at[c1]], add=True)
            return c2

        lax.fori_loop(0, npair, pair, 0)

        c_last = nch - 1
        wait_in(c_last, buf0, si0)
        pltpu.sync_copy(buf0, agg.at[idx.at[c_last]], add=True)
        plsc.subcore_barrier()

        @pl.when(sid == 0)
        def _out():
            pltpu.sync_copy(agg, out_hbm.at[cid])

    f = pl.kernel(
        body,
        out_type=jax.ShapeDtypeStruct((NC, n, hid), jnp.float32),
        mesh=mesh,
        scratch_types=[
            pltpu.VMEM((nch, CH), jnp.int32),
            pltpu.VMEM((CH, hid), jnp.float32),
            pltpu.VMEM((CH, hid), jnp.float32),
            pltpu.SemaphoreType.DMA,
            pltpu.SemaphoreType.DMA,
            pltpu.VMEM_SHARED((n, hid), jnp.float32),
        ],
    )
    return f(mij, row3d, zeros_nh)


def _k5_body(h_ref, p_ref, w3h_ref, w3a_ref, b3_ref, w4_ref, b4_ref, o_ref):
    hb = h_ref[...]
    a = (p_ref[0] + p_ref[1]) * (1.0 / NORM)
    y = (jnp.dot(hb, w3h_ref[...], preferred_element_type=jnp.float32)
         + jnp.dot(a, w3a_ref[...], preferred_element_type=jnp.float32)
         + b3_ref[...])
    y = _silu(y)
    o_ref[...] = hb + jnp.dot(y, w4_ref[...], preferred_element_type=jnp.float32) + b4_ref[...]


def _k5(h, aggp, w3h, w3a, b3, w4, b4, bn):
    n, nf = h.shape
    hid = w3h.shape[1]
    return pl.pallas_call(
        _k5_body,
        grid=(n // bn,),
        in_specs=[
            pl.BlockSpec((bn, nf), lambda i: (i, 0)),
            pl.BlockSpec((NC, bn, hid), lambda i: (0, i, 0)),
            pl.BlockSpec((nf, hid), lambda i: (0, 0)),
            pl.BlockSpec((hid, hid), lambda i: (0, 0)),
            pl.BlockSpec((1, hid), lambda i: (0, 0)),
            pl.BlockSpec((hid, nf), lambda i: (0, 0)),
            pl.BlockSpec((1, nf), lambda i: (0, 0)),
        ],
        out_specs=pl.BlockSpec((bn, nf), lambda i: (i, 0)),
        out_shape=jax.ShapeDtypeStruct((n, nf), jnp.float32),
    )(h, aggp, w3h, w3a, b3, w4, b4)


def kernel(h, edge_index, edge_attr, W1, b1, W2, b2, W3, b3, W4, b4):
    n, nf = h.shape
    e, ea = edge_attr.shape
    hid = W2.shape[0]

    row = edge_index[0].astype(jnp.int32)
    col = edge_index[1].astype(jnp.int32)
    nch = e // (NW * CH)
    row3d = row.reshape(NW, nch, CH)
    col3d = col.reshape(NW, nch, CH)

    w1s = W1[:nf]
    w1t = W1[nf:2 * nf]
    w1e = W1[2 * nf:]

    g1, g2 = _k1(h, w1s, w1t, 1000)
    s = _sc_gather_sum(g1, g2, row3d, col3d, e, hid)
    mij = _k3(s, edge_attr, w1e, b1.reshape(1, hid), W2, b2.reshape(1, hid), 2000)
    aggp = _sc_segsum(mij, row3d, jnp.zeros((n, hid), jnp.float32), n, e, hid)
    h_out = _k5(h, aggp, W3[:nf], W3[nf:], b3.reshape(1, hid), W4,
                b4.reshape(1, nf), 1000)
    return (h_out, mij)


# speedup vs baseline: 5.0660x; 1.0888x over previous
import functools

import jax
import jax.numpy as jnp
from jax import lax
from jax.experimental import pallas as pl
from jax.experimental.pallas import tpu as pltpu
from jax.experimental.pallas import tpu_sc as plsc

NC = 2
NS = 16
NW = NC * NS

CH = 80
LANES = 16
NORM = 100.0


def _silu(x):
    return x * jax.nn.sigmoid(x)


def _k1_body(h_ref, w1s_ref, w1t_ref, o1_ref, o2_ref):
    hb = h_ref[...]
    o1_ref[...] = jnp.dot(hb, w1s_ref[...], preferred_element_type=jnp.float32)
    o2_ref[...] = jnp.dot(hb, w1t_ref[...], preferred_element_type=jnp.float32)


def _k1(h, w1s, w1t, bn):
    n, nf = h.shape
    hid = w1s.shape[1]
    return pl.pallas_call(
        _k1_body,
        grid=(n // bn,),
        in_specs=[
            pl.BlockSpec((bn, nf), lambda i: (i, 0)),
            pl.BlockSpec((nf, hid), lambda i: (0, 0)),
            pl.BlockSpec((nf, hid), lambda i: (0, 0)),
        ],
        out_specs=[
            pl.BlockSpec((bn, hid), lambda i: (i, 0)),
            pl.BlockSpec((bn, hid), lambda i: (i, 0)),
        ],
        out_shape=[
            jax.ShapeDtypeStruct((n, hid), jnp.float32),
            jax.ShapeDtypeStruct((n, hid), jnp.float32),
        ],
    )(h, w1s, w1t)


def _sc_gather_sum(g1, g2, row3d, col3d, e, hid, CH):
    epw = e // NW
    nch = epw // CH
    mesh = plsc.VectorSubcoreMesh(
        core_axis_name="c", subcore_axis_name="s",
        num_cores=NC, num_subcores=NS)

    assert nch % 2 == 1 and nch >= 3
    npair = (nch - 1) // 2

    def body(g1_hbm, g2_hbm, row_hbm, col_hbm, out_hbm,
             idx_r, idx_c, ba0, bb0, ba1, bb1, sg0, sg1, so0, so1):
        cid = lax.axis_index("c")
        sid = lax.axis_index("s")
        wid = sid * NC + cid
        base = wid * epw
        pltpu.sync_copy(row_hbm.at[wid], idx_r)
        pltpu.sync_copy(col_hbm.at[wid], idx_c)

        def fire(c, ba, bb, sg):
            pltpu.async_copy(g1_hbm.at[idx_r.at[c]], ba, sg)
            pltpu.async_copy(g2_hbm.at[idx_c.at[c]], bb, sg)

        def wait_gather(c, ba, bb, sg):
            pltpu.make_async_copy(g1_hbm.at[idx_r.at[c]], ba, sg).wait()
            pltpu.make_async_copy(g2_hbm.at[idx_c.at[c]], bb, sg).wait()

        def add(ba, bb):
            def add_row(r, c2):
                for t in range(hid // LANES):
                    sl = pl.ds(t * LANES, LANES)
                    ba[r, sl] = ba[r, sl] + bb[r, sl]
                return c2

            lax.fori_loop(0, CH, add_row, 0)

        def store(c, ba, so):
            pltpu.async_copy(ba, out_hbm.at[pl.ds(base + c * CH, CH)], so)

        def wait_store(c, ba, so):
            pltpu.make_async_copy(ba, out_hbm.at[pl.ds(base + c * CH, CH)], so).wait()

        fire(0, ba0, bb0, sg0)

        def pair(i2, carry):
            c0 = 2 * i2
            c1 = c0 + 1
            fire(c1, ba1, bb1, sg1)
            wait_gather(c0, ba0, bb0, sg0)

            @pl.when(i2 > 0)
            def _w0():
                wait_store(c0 - 2, ba0, so0)

            add(ba0, bb0)
            store(c0, ba0, so0)

            fire(c0 + 2, ba0, bb0, sg0)
            wait_gather(c1, ba1, bb1, sg1)

            @pl.when(i2 > 0)
            def _w1():
                wait_store(c1 - 2, ba1, so1)

            add(ba1, bb1)
            store(c1, ba1, so1)
            return carry

        lax.fori_loop(0, npair, pair, 0)

        c_last = nch - 1
        wait_gather(c_last, ba0, bb0, sg0)
        wait_store(c_last - 2, ba0, so0)
        add(ba0, bb0)
        store(c_last, ba0, so0)
        wait_store(c_last - 1, ba1, so1)
        wait_store(c_last, ba0, so0)

    f = pl.kernel(
        body,
        out_type=jax.ShapeDtypeStruct((e, hid), jnp.float32),
        mesh=mesh,
        scratch_types=[
            pltpu.VMEM((nch, CH), jnp.int32),
            pltpu.VMEM((nch, CH), jnp.int32),
            pltpu.VMEM((CH, hid), jnp.float32),
            pltpu.VMEM((CH, hid), jnp.float32),
            pltpu.VMEM((CH, hid), jnp.float32),
            pltpu.VMEM((CH, hid), jnp.float32),
            pltpu.SemaphoreType.DMA,
            pltpu.SemaphoreType.DMA,
            pltpu.SemaphoreType.DMA,
            pltpu.SemaphoreType.DMA,
        ],
    )
    return f(g1, g2, row3d, col3d)


def _k3_body(s_ref, ea_ref, w1e_ref, b1_ref, w2_ref, b2_ref, o_ref):
    x = (s_ref[...]
         + jnp.dot(ea_ref[...], w1e_ref[...], preferred_element_type=jnp.float32)
         + b1_ref[...])
    x = _silu(x)
    y = jnp.dot(x, w2_ref[...], preferred_element_type=jnp.float32) + b2_ref[...]
    o_ref[...] = _silu(y)


def _k3(s, edge_attr, w1e, b1, w2, b2, be):
    e, hid = s.shape
    ea = edge_attr.shape[1]
    return pl.pallas_call(
        _k3_body,
        grid=(e // be,),
        in_specs=[
            pl.BlockSpec((be, hid), lambda i: (i, 0)),
            pl.BlockSpec((be, ea), lambda i: (i, 0)),
            pl.BlockSpec((ea, hid), lambda i: (0, 0)),
            pl.BlockSpec((1, hid), lambda i: (0, 0)),
            pl.BlockSpec((hid, hid), lambda i: (0, 0)),
            pl.BlockSpec((1, hid), lambda i: (0, 0)),
        ],
        out_specs=pl.BlockSpec((be, hid), lambda i: (i, 0)),
        out_shape=jax.ShapeDtypeStruct((e, hid), jnp.float32),
    )(s, edge_attr, w1e, b1, w2, b2)


def _sc_segsum(mij, row3d, zeros_nh, n, e, hid, CH):
    epw = e // NW
    nch = epw // CH
    assert nch % 2 == 1 and nch >= 3
    npair = (nch - 1) // 2
    mesh = plsc.VectorSubcoreMesh(
        core_axis_name="c", subcore_axis_name="s",
        num_cores=NC, num_subcores=NS)

    def body(mij_hbm, row_hbm, z_hbm, out_hbm, idx, buf0, buf1, si0, si1, agg):
        cid = lax.axis_index("c")
        sid = lax.axis_index("s")
        wid = sid * NC + cid

        @pl.when(sid == 0)
        def _zero():
            pltpu.sync_copy(z_hbm, agg)

        plsc.subcore_barrier()

        pltpu.sync_copy(row_hbm.at[wid], idx)
        base = wid * epw

        def fire_in(c, buf, si):
            pltpu.async_copy(mij_hbm.at[pl.ds(base + c * CH, CH)], buf, si)

        def wait_in(c, buf, si):
            pltpu.make_async_copy(mij_hbm.at[pl.ds(base + c * CH, CH)], buf, si).wait()

        fire_in(0, buf0, si0)

        def pair(i2, c2):
            c0 = 2 * i2
            c1 = c0 + 1
            fire_in(c1, buf1, si1)
            wait_in(c0, buf0, si0)
            pltpu.sync_copy(buf0, agg.at[idx.at[c0]], add=True)
            fire_in(c0 + 2, buf0, si0)
            wait_in(c1, buf1, si1)
            pltpu.sync_copy(buf1, agg.at[idx.at[c1]], add=True)
            return c2

        lax.fori_loop(0, npair, pair, 0)

        c_last = nch - 1
        wait_in(c_last, buf0, si0)
        pltpu.sync_copy(buf0, agg.at[idx.at[c_last]], add=True)
        plsc.subcore_barrier()

        @pl.when(sid == 0)
        def _out():
            pltpu.sync_copy(agg, out_hbm.at[cid])

    f = pl.kernel(
        body,
        out_type=jax.ShapeDtypeStruct((NC, n, hid), jnp.float32),
        mesh=mesh,
        scratch_types=[
            pltpu.VMEM((nch, CH), jnp.int32),
            pltpu.VMEM((CH, hid), jnp.float32),
            pltpu.VMEM((CH, hid), jnp.float32),
            pltpu.SemaphoreType.DMA,
            pltpu.SemaphoreType.DMA,
            pltpu.VMEM_SHARED((n, hid), jnp.float32),
        ],
    )
    return f(mij, row3d, zeros_nh)


def _k5_body(h_ref, p_ref, w3h_ref, w3a_ref, b3_ref, w4_ref, b4_ref, o_ref):
    hb = h_ref[...]
    a = (p_ref[0] + p_ref[1]) * (1.0 / NORM)
    y = (jnp.dot(hb, w3h_ref[...], preferred_element_type=jnp.float32)
         + jnp.dot(a, w3a_ref[...], preferred_element_type=jnp.float32)
         + b3_ref[...])
    y = _silu(y)
    o_ref[...] = hb + jnp.dot(y, w4_ref[...], preferred_element_type=jnp.float32) + b4_ref[...]


def _k5(h, aggp, w3h, w3a, b3, w4, b4, bn):
    n, nf = h.shape
    hid = w3h.shape[1]
    return pl.pallas_call(
        _k5_body,
        grid=(n // bn,),
        in_specs=[
            pl.BlockSpec((bn, nf), lambda i: (i, 0)),
            pl.BlockSpec((NC, bn, hid), lambda i: (0, i, 0)),
            pl.BlockSpec((nf, hid), lambda i: (0, 0)),
            pl.BlockSpec((hid, hid), lambda i: (0, 0)),
            pl.BlockSpec((1, hid), lambda i: (0, 0)),
            pl.BlockSpec((hid, nf), lambda i: (0, 0)),
            pl.BlockSpec((1, nf), lambda i: (0, 0)),
        ],
        out_specs=pl.BlockSpec((bn, nf), lambda i: (i, 0)),
        out_shape=jax.ShapeDtypeStruct((n, nf), jnp.float32),
    )(h, aggp, w3h, w3a, b3, w4, b4)


def kernel(h, edge_index, edge_attr, W1, b1, W2, b2, W3, b3, W4, b4):
    n, nf = h.shape
    e, ea = edge_attr.shape
    hid = W2.shape[0]

    row = edge_index[0].astype(jnp.int32)
    col = edge_index[1].astype(jnp.int32)

    w1s = W1[:nf]
    w1t = W1[nf:2 * nf]
    w1e = W1[2 * nf:]
    b1r = b1.reshape(1, hid)
    b2r = b2.reshape(1, hid)
    zeros_nh = jnp.zeros((n, hid), jnp.float32)

    ch = 80
    nch = e // (NW * ch)
    row3d = row.reshape(NW, nch, ch)
    col3d = col.reshape(NW, nch, ch)

    g1, g2 = _k1(h, w1s, w1t, 1000)
    s = _sc_gather_sum(g1, g2, row3d, col3d, e, hid, ch)
    mij = _k3(s, edge_attr, w1e, b1r, W2, b2r, 4000)
    aggp = _sc_segsum(mij, row3d, zeros_nh, n, e, hid, ch)
    h_out = _k5(h, aggp, W3[:nf], W3[nf:], b3.reshape(1, hid), W4,
                b4.reshape(1, nf), 1000)
    return (h_out, mij)
